# Initial kernel scaffold; baseline (speedup 1.0000x reference)
import jax
import jax.numpy as jnp
from jax.experimental import pallas as pl


def kernel(features, nl_features, W, a, adj, target_index_out, neighbor_index_target, target_len, neighbor_len):
    raise NotImplementedError("write your pallas kernel here")


# speedup vs baseline: 6.1209x; 6.1209x over previous
import functools

import jax
import jax.numpy as jnp
from jax import lax
from jax.experimental import pallas as pl
from jax.experimental.pallas import tpu as pltpu
from jax.experimental.pallas import tpu_sc as plsc

N = 10000
E = 160000
D = 128
ALPHA = 0.2
NEG = -3.0e38

NW = 32
RPT = 320
SUB = 80
CH = 1600
NCH = E // CH
L = 6144
LPAD = L + 128
K = 128


def _tc_body(feat, nlf, w, a1, a2, h, g, s1, s2):
    hh = jnp.dot(feat[...], w[...], preferred_element_type=jnp.float32)
    gg = jnp.dot(nlf[...], w[...], preferred_element_type=jnp.float32)
    h[...] = hh
    g[...] = gg
    s1[...] = jnp.dot(hh, a1[...], preferred_element_type=jnp.float32)
    s2[...] = jnp.dot(hh, a2[...], preferred_element_type=jnp.float32)


def _tc_precompute(features, nl_features, W, a1, a2):
    blk = 1000
    grid = (N // blk,)
    return pl.pallas_call(
        _tc_body,
        grid=grid,
        in_specs=[
            pl.BlockSpec((blk, D), lambda i: (i, 0)),
            pl.BlockSpec((blk, D), lambda i: (i, 0)),
            pl.BlockSpec((D, D), lambda i: (0, 0)),
            pl.BlockSpec((D, 1), lambda i: (0, 0)),
            pl.BlockSpec((D, 1), lambda i: (0, 0)),
        ],
        out_specs=[
            pl.BlockSpec((blk, D), lambda i: (i, 0)),
            pl.BlockSpec((blk, D), lambda i: (i, 0)),
            pl.BlockSpec((blk, 1), lambda i: (i, 0)),
            pl.BlockSpec((blk, 1), lambda i: (i, 0)),
        ],
        out_shape=[
            jax.ShapeDtypeStruct((N, D), jnp.float32),
            jax.ShapeDtypeStruct((N, D), jnp.float32),
            jax.ShapeDtypeStruct((N, 1), jnp.float32),
            jax.ShapeDtypeStruct((N, 1), jnp.float32),
        ],
    )(features, nl_features, W, a1, a2)


def _sc_body(h_hbm, g_hbm, s1_hbm, s2_hbm, a2_hbm, adj0_hbm, adj1_hbm,
             tio_hbm, nit_hbm, out1_hbm, out2_hbm,
             s1b, s2b, a2b, eb0a, eb1a, ebta, eb0b, eb1b, ebtb,
             lrow, lcol, lval,
             crow, ccol, cval, tmp, cnt, base, fill, nitb, nleb, mb, zb,
             idxs, acc, hbuf, nlbuf, o1b,
             semE0, semE1, semH, semN):
    cid = lax.axis_index("c")
    sid = lax.axis_index("s")
    wid = sid * 2 + cid
    r0 = pl.multiple_of(wid * RPT, 8)
    last = wid == NW - 1
    rows_this = jnp.where(last, N - (NW - 1) * RPT, RPT)

    iota = lax.iota(jnp.int32, 16)
    lane0 = iota == 0
    zeros16 = jnp.zeros((16,), jnp.float32)
    ones16i = jnp.ones((16,), jnp.int32)

    def sload(ref, i):
        return plsc.load_gather(ref, [jnp.full((16,), i, jnp.int32)])[0]

    def sstore(ref, i, v):
        plsc.store_scatter(ref, [jnp.full((16,), i, jnp.int32)],
                           jnp.full((16,), v, ref.dtype), mask=lane0)

    pltpu.sync_copy(s1_hbm, s1b)
    pltpu.sync_copy(s2_hbm, s2b)
    pltpu.sync_copy(a2_hbm, a2b)

    tail_rows = N - (NW - 1) * RPT

    @pl.when(jnp.logical_not(last))
    def _():
        pltpu.sync_copy(nit_hbm.at[pl.ds(r0, RPT)], nitb)

    @pl.when(last)
    def _():
        pltpu.sync_copy(nit_hbm.at[pl.ds(r0, tail_rows)],
                        nitb.at[pl.ds(0, tail_rows)])

    def _zero_cnt(i, _):
        cnt[pl.ds(i * 16, 16)] = jnp.zeros((16,), jnp.int32)
        return 0
    lax.fori_loop(0, RPT // 16, _zero_cnt, 0)

    def _zero_ccol(i, _):
        ccol[pl.ds(i * 16, 16)] = jnp.zeros((16,), jnp.int32)
        return 0
    lax.fori_loop(0, LPAD // 16, _zero_ccol, 0)

    bufs = ((eb0a, eb1a, ebta), (eb0b, eb1b, ebtb))

    def _fire(c, buf, sem):
        off = pl.multiple_of(c * CH, 8)
        pltpu.async_copy(adj0_hbm.at[pl.ds(off, CH)], buf[0], sem)
        pltpu.async_copy(adj1_hbm.at[pl.ds(off, CH)], buf[1], sem)
        pltpu.async_copy(tio_hbm.at[pl.ds(off, CH)], buf[2], sem)

    def _drain(c, buf, sem):
        off = pl.multiple_of(c * CH, 8)
        pltpu.make_async_copy(adj0_hbm.at[pl.ds(off, CH)], buf[0], sem).wait()
        pltpu.make_async_copy(adj1_hbm.at[pl.ds(off, CH)], buf[1], sem).wait()
        pltpu.make_async_copy(tio_hbm.at[pl.ds(off, CH)], buf[2], sem).wait()

    def _scan_chunk(buf, ptr):
        def _grp(gi, ptr):
            t = buf[2][pl.ds(gi * 16, 16)]
            a0 = buf[0][pl.ds(gi * 16, 16)]
            a1 = buf[1][pl.ds(gi * 16, 16)]
            msk = jnp.logical_and(t >= r0, t < r0 + RPT)
            sv = (plsc.load_gather(s1b, [a0])
                  + plsc.load_gather(s2b, [a1]))
            ev = jnp.where(sv >= 0.0, sv, sv * ALPHA)
            plsc.addupdate_scatter(cnt, [t - r0], ones16i, mask=msk)
            plsc.store_compressed(lrow.at[pl.ds(ptr, 16)], t - r0, mask=msk)
            plsc.store_compressed(lcol.at[pl.ds(ptr, 16)], a1, mask=msk)
            plsc.store_compressed(lval.at[pl.ds(ptr, 16)], ev, mask=msk)
            nhit = jnp.sum(msk.astype(jnp.int32))
            return jnp.minimum(ptr + nhit, L - 16)
        return lax.fori_loop(0, CH // 16, _grp, ptr)

    _fire(0, bufs[0], semE0)

    def _pair(cc, ptr):
        c0 = cc * 2
        _fire(c0 + 1, bufs[1], semE1)
        _drain(c0, bufs[0], semE0)
        ptr = _scan_chunk(bufs[0], ptr)

        @pl.when(c0 + 2 < NCH)
        def _():
            _fire(c0 + 2, bufs[0], semE0)
        _drain(c0 + 1, bufs[1], semE1)
        ptr = _scan_chunk(bufs[1], ptr)
        return ptr
    local_cnt = lax.fori_loop(0, NCH // 2, _pair, jnp.int32(0))

    def _pfx(i, carry):
        v = cnt[pl.ds(i * 16, 16)]
        ex = plsc.cumsum(v) - v
        bv = ex + carry
        base[pl.ds(i * 16, 16)] = bv
        fill[pl.ds(i * 16, 16)] = bv
        return carry + jnp.sum(v)
    lax.fori_loop(0, RPT // 16, _pfx, jnp.int32(0))

    def _place(j, _):
        r = sload(lrow, j)
        p = sload(fill, r)
        sstore(crow, p, r)
        sstore(ccol, p, sload(lcol, j))
        sstore(cval, p, sload(lval, j))
        sstore(fill, r, p + 1)
        return 0
    lax.fori_loop(0, local_cnt, _place, 0)

    def _nl_group(gi, _):
        gb = gi * 16
        pltpu.async_copy(g_hbm.at[nitb.at[pl.ds(gb, 16)]], nlbuf, semN).wait()
        pv = zeros16
        for rr in range(16):
            v = nlbuf[rr, pl.ds(0, 16)] * a2b[pl.ds(0, 16)]
            for u in range(1, 8):
                v = v + nlbuf[rr, pl.ds(u * 16, 16)] * a2b[pl.ds(u * 16, 16)]
            pv = jnp.where(iota == rr, jnp.full((16,), jnp.sum(v)), pv)
        nitv = nitb[pl.ds(gb, 16)]
        sg = plsc.load_gather(s1b, [nitv])
        x = sg + pv
        nleb[pl.ds(gb, 16)] = jnp.where(x >= 0.0, x, x * ALPHA)
        for rr in range(16):
            for u in range(8):
                y = nlbuf[rr, pl.ds(u * 16, 16)]
                nlbuf[rr, pl.ds(u * 16, 16)] = jnp.where(
                    y > 0.0, y, jnp.exp(y) - 1.0)
        pltpu.sync_copy(nlbuf, out2_hbm.at[pl.ds(r0 + gb, 16)])
        return 0
    lax.fori_loop(0, rows_this // 16, _nl_group, 0)

    def _row(r, _):
        lo = sload(base, r)
        hi = sload(fill, r)

        def _tag(p, _):
            colv = plsc.load_gather(ccol, [jnp.full((16,), p, jnp.int32)])
            plsc.store_scatter(tmp, [colv],
                               jnp.full((16,), p, jnp.int32), mask=lane0)
            return 0
        lax.fori_loop(lo, hi, _tag, 0)

        nch = (hi - lo + 15) // 16

        def _mx(ch, m):
            idxv = lo + ch * 16 + iota
            inb = idxv < hi
            cols = plsc.load_gather(ccol, [idxv])
            win = plsc.load_gather(tmp, [cols])
            valid = jnp.logical_and(inb, win == idxv)
            ev = plsc.load_gather(cval, [idxv])
            return jnp.maximum(m, jnp.max(jnp.where(valid, ev, NEG)))
        m = lax.fori_loop(0, nch, _mx, jnp.float32(NEG))
        m = jnp.maximum(m, sload(nleb, r))

        def _ex(ch, z):
            idxv = lo + ch * 16 + iota
            inb = idxv < hi
            cols = plsc.load_gather(ccol, [idxv])
            win = plsc.load_gather(tmp, [cols])
            valid = jnp.logical_and(inb, win == idxv)
            ev = plsc.load_gather(cval, [idxv])
            w = jnp.where(valid, jnp.exp(ev - m), 0.0)
            plsc.store_scatter(cval, [idxv], w, mask=inb)
            return z + jnp.sum(w)
        z = lax.fori_loop(0, nch, _ex, jnp.float32(0.0))
        sstore(mb, r, m)
        sstore(zb, r, z)
        return 0
    lax.fori_loop(0, rows_this, _row, 0)

    def _sub(s, _):
        sr0 = s * SUB
        lo = sload(base, sr0)
        hi = sload(fill, sr0 + SUB - 1)

        def _zacc(r, _):
            for u in range(8):
                acc[r, pl.ds(u * 16, 16)] = zeros16
            return 0
        lax.fori_loop(0, SUB, _zacc, 0)

        nchunks = (hi - lo + K - 1) // K

        def _chunk(ci, _):
            c = lo + ci * K
            for u in range(8):
                idxs[pl.ds(u * 16, 16)] = plsc.load_gather(
                    ccol, [c + u * 16 + iota])
            pltpu.async_copy(h_hbm.at[idxs], hbuf, semH).wait()
            nloc = jnp.minimum(K, hi - c)

            def _edge(jj, _):
                r = sload(crow, c + jj)
                w = sload(cval, c + jj)
                rs = r - sr0
                wv = jnp.full((16,), w, jnp.float32)
                for u in range(8):
                    acc[rs, pl.ds(u * 16, 16)] = (
                        acc[rs, pl.ds(u * 16, 16)]
                        + wv * hbuf[jj, pl.ds(u * 16, 16)])
                return 0
            lax.fori_loop(0, nloc, _edge, 0)
            return 0
        lax.fori_loop(0, nchunks, _chunk, 0)

        def _fin_group(gi, _):
            gb = sr0 + gi * 16
            pltpu.async_copy(g_hbm.at[nitb.at[pl.ds(gb, 16)]], nlbuf, semN).wait()
            nlev = nleb[pl.ds(gb, 16)]
            mv = mb[pl.ds(gb, 16)]
            zv = zb[pl.ds(gb, 16)]
            wun = jnp.exp(nlev - mv)
            zt = zv + wun
            iz = 1.0 / zt
            wn = wun * iz
            for rr in range(16):
                rs = gi * 16 + rr
                izv = jnp.full((16,), iz[rr], jnp.float32)
                wnv = jnp.full((16,), wn[rr], jnp.float32)
                for u in range(8):
                    x = (acc[rs, pl.ds(u * 16, 16)] * izv
                         + wnv * nlbuf[rr, pl.ds(u * 16, 16)])
                    o1b[rr, pl.ds(u * 16, 16)] = jnp.where(
                        x > 0.0, x, jnp.exp(x) - 1.0)
            pltpu.sync_copy(o1b, out1_hbm.at[pl.ds(r0 + gb, 16)])
            return 0
        lax.fori_loop(0, SUB // 16, _fin_group, 0)
        return 0
    lax.fori_loop(0, rows_this // SUB, _sub, 0)


def _sc_attention(h, g, s1, s2, a2v, adj0, adj1, tio, nit):
    mesh = plsc.VectorSubcoreMesh(core_axis_name="c", subcore_axis_name="s")
    f32 = jnp.float32
    i32 = jnp.int32
    kern = pl.kernel(
        _sc_body,
        out_type=[
            jax.ShapeDtypeStruct((N, D), f32),
            jax.ShapeDtypeStruct((N, D), f32),
        ],
        mesh=mesh,
        compiler_params=pltpu.CompilerParams(needs_layout_passes=False),
        scratch_types=[
            pltpu.VMEM((N,), f32),
            pltpu.VMEM((N,), f32),
            pltpu.VMEM((D,), f32),
            pltpu.VMEM((CH,), i32),
            pltpu.VMEM((CH,), i32),
            pltpu.VMEM((CH,), i32),
            pltpu.VMEM((CH,), i32),
            pltpu.VMEM((CH,), i32),
            pltpu.VMEM((CH,), i32),
            pltpu.VMEM((L,), i32),
            pltpu.VMEM((L,), i32),
            pltpu.VMEM((L,), f32),
            pltpu.VMEM((LPAD,), i32),
            pltpu.VMEM((LPAD,), i32),
            pltpu.VMEM((LPAD,), f32),
            pltpu.VMEM((N,), i32),
            pltpu.VMEM((RPT,), i32),
            pltpu.VMEM((RPT,), i32),
            pltpu.VMEM((RPT,), i32),
            pltpu.VMEM((RPT,), i32),
            pltpu.VMEM((RPT,), f32),
            pltpu.VMEM((RPT,), f32),
            pltpu.VMEM((RPT,), f32),
            pltpu.VMEM((K,), i32),
            pltpu.VMEM((SUB, D), f32),
            pltpu.VMEM((K, D), f32),
            pltpu.VMEM((16, D), f32),
            pltpu.VMEM((16, D), f32),
            pltpu.SemaphoreType.DMA,
            pltpu.SemaphoreType.DMA,
            pltpu.SemaphoreType.DMA,
            pltpu.SemaphoreType.DMA,
        ],
    )
    return kern(h, g, s1, s2, a2v, adj0, adj1, tio, nit)


@jax.jit
def kernel(features, nl_features, W, a, adj, target_index_out,
           neighbor_index_target, target_len, neighbor_len):
    a1 = a[:D]
    a2 = a[D:]
    h, g, s1, s2 = _tc_precompute(features, nl_features, W, a1, a2)
    out1, out2 = _sc_attention(
        h, g, s1.reshape(-1), s2.reshape(-1), a2.reshape(-1),
        adj[0], adj[1], target_index_out, neighbor_index_target)
    return (out1, out2)
